# Initial kernel scaffold; baseline (speedup 1.0000x reference)
import functools

import jax
import jax.numpy as jnp
from jax import lax
from jax.experimental import pallas as pl
from jax.experimental.pallas import tpu as pltpu
from jax.experimental.pallas import tpu_sc as plsc

NC = 2
NS = 16
NW = NC * NS

BATCH = 1024
NEG = 200
DIM = 64

ENT_ROWS = BATCH * (NEG + 2)
REL_ROWS = BATCH
ENT_PER_W = ENT_ROWS // NW
REL_PER_W = REL_ROWS // NW
CHUNK = 64
N_CHUNKS = ENT_PER_W // CHUNK


def _body(ent_idx_hbm, rel_idx_hbm, oe_hbm, orl_hbm, e_hbm, rl_hbm,
          out_oe, out_orl, out_e, out_rl,
          idx_v, ridx_v, buf_a, buf_b, rbuf_a, rbuf_b, sem_a, sem_b):
  wid = lax.axis_index("s") * NC + lax.axis_index("c")

  pltpu.sync_copy(ent_idx_hbm.at[wid], idx_v)
  pltpu.sync_copy(rel_idx_hbm.at[wid], ridx_v)

  cp_a = pltpu.async_copy(orl_hbm.at[ridx_v], rbuf_a, sem_a)
  cp_b = pltpu.async_copy(rl_hbm.at[ridx_v], rbuf_b, sem_b)
  cp_a.wait()
  cp_b.wait()
  rbase = wid * REL_PER_W
  pltpu.sync_copy(rbuf_a, out_orl.at[pl.ds(rbase, REL_PER_W)])
  pltpu.sync_copy(rbuf_b, out_rl.at[pl.ds(rbase, REL_PER_W)])

  base = wid * ENT_PER_W

  def chunk(j, carry):
    g_a = pltpu.async_copy(oe_hbm.at[idx_v.at[j]], buf_a, sem_a)
    g_b = pltpu.async_copy(e_hbm.at[idx_v.at[j]], buf_b, sem_b)
    g_a.wait()
    g_b.wait()
    off = base + j * CHUNK
    pltpu.sync_copy(buf_a, out_oe.at[pl.ds(off, CHUNK)])
    pltpu.sync_copy(buf_b, out_e.at[pl.ds(off, CHUNK)])
    return carry

  lax.fori_loop(0, N_CHUNKS, chunk, 0)


@jax.jit
def _gather_all(ent_idx, rel_idx, oe, orl, e, rl):
  mesh = plsc.VectorSubcoreMesh(core_axis_name="c", subcore_axis_name="s")
  f32 = jnp.float32
  run = functools.partial(
      pl.kernel,
      out_type=(
          jax.ShapeDtypeStruct((ENT_ROWS, DIM), f32),
          jax.ShapeDtypeStruct((REL_ROWS, DIM), f32),
          jax.ShapeDtypeStruct((ENT_ROWS, DIM), f32),
          jax.ShapeDtypeStruct((REL_ROWS, DIM), f32),
      ),
      mesh=mesh,
      scratch_types=[
          pltpu.VMEM((N_CHUNKS, CHUNK), jnp.int32),
          pltpu.VMEM((REL_PER_W,), jnp.int32),
          pltpu.VMEM((CHUNK, DIM), f32),
          pltpu.VMEM((CHUNK, DIM), f32),
          pltpu.VMEM((REL_PER_W, DIM), f32),
          pltpu.VMEM((REL_PER_W, DIM), f32),
          pltpu.SemaphoreType.DMA,
          pltpu.SemaphoreType.DMA,
      ],
  )(_body)
  return run(ent_idx, rel_idx, oe, orl, e, rl)


def kernel(positive, negative, origin_entity_embedding,
           origin_relation_embedding, entity_embedding, relation_embedding):
  tail_idx = jnp.concatenate([positive[:, 2:3], negative], axis=1).reshape(-1)
  ent_idx = jnp.concatenate([positive[:, 0], tail_idx]).reshape(NW, N_CHUNKS, CHUNK)
  rel_idx = positive[:, 1].reshape(NW, REL_PER_W)

  out_oe, out_orl, out_e, out_rl = _gather_all(
      ent_idx.astype(jnp.int32), rel_idx.astype(jnp.int32),
      origin_entity_embedding, origin_relation_embedding,
      entity_embedding, relation_embedding)

  origin_head = out_oe[:BATCH].reshape(BATCH, 1, DIM)
  origin_tail = out_oe[BATCH:].reshape(BATCH, NEG + 1, DIM)
  head = out_e[:BATCH].reshape(BATCH, 1, DIM)
  tail = out_e[BATCH:].reshape(BATCH, NEG + 1, DIM)
  origin_relation = out_orl.reshape(BATCH, 1, DIM)
  relation = out_rl.reshape(BATCH, 1, DIM)
  return (origin_head, origin_relation, origin_tail, head, relation, tail)


# speedup vs baseline: 2.2224x; 2.2224x over previous
import functools

import jax
import jax.numpy as jnp
from jax import lax
from jax.experimental import pallas as pl
from jax.experimental.pallas import tpu as pltpu
from jax.experimental.pallas import tpu_sc as plsc

NC = 2
NS = 16
NW = NC * NS

BATCH = 1024
NEG = 200
DIM = 64

ENT_ROWS = BATCH * (NEG + 2)
REL_ROWS = BATCH
ENT_PER_W = ENT_ROWS // NW
REL_PER_W = REL_ROWS // NW
CHUNK = 64
N_CHUNKS = ENT_PER_W // CHUNK


def _body(ent_idx_hbm, rel_idx_hbm, oe_hbm, orl_hbm, e_hbm, rl_hbm,
          out_oe, out_orl, out_e, out_rl,
          idx_v, ridx_v, buf_a, buf_b, rbuf_a, rbuf_b, sem_a, sem_b):
  wid = lax.axis_index("s") * NC + lax.axis_index("c")

  pltpu.sync_copy(ent_idx_hbm.at[wid], idx_v)
  pltpu.sync_copy(rel_idx_hbm.at[wid], ridx_v)

  cp_a = pltpu.async_copy(orl_hbm.at[ridx_v], rbuf_a, sem_a)
  cp_b = pltpu.async_copy(rl_hbm.at[ridx_v], rbuf_b, sem_b)
  cp_a.wait()
  cp_b.wait()
  rbase = wid * REL_PER_W
  pltpu.sync_copy(rbuf_a, out_orl.at[pl.ds(rbase, REL_PER_W)])
  pltpu.sync_copy(rbuf_b, out_rl.at[pl.ds(rbase, REL_PER_W)])

  base = wid * ENT_PER_W

  def chunk(j, carry):
    g_a = pltpu.async_copy(oe_hbm.at[idx_v.at[j]], buf_a, sem_a)
    g_b = pltpu.async_copy(e_hbm.at[idx_v.at[j]], buf_b, sem_b)
    g_a.wait()
    g_b.wait()
    off = base + j * CHUNK
    pltpu.sync_copy(buf_a, out_oe.at[pl.ds(off, CHUNK)])
    pltpu.sync_copy(buf_b, out_e.at[pl.ds(off, CHUNK)])
    return carry

  lax.fori_loop(0, N_CHUNKS, chunk, 0)


@jax.jit
def _gather_all(ent_idx, rel_idx, oe, orl, e, rl):
  mesh = plsc.VectorSubcoreMesh(core_axis_name="c", subcore_axis_name="s")
  f32 = jnp.float32
  run = functools.partial(
      pl.kernel,
      out_type=(
          jax.ShapeDtypeStruct((ENT_ROWS, DIM), f32),
          jax.ShapeDtypeStruct((REL_ROWS, DIM), f32),
          jax.ShapeDtypeStruct((ENT_ROWS, DIM), f32),
          jax.ShapeDtypeStruct((REL_ROWS, DIM), f32),
      ),
      mesh=mesh,
      compiler_params=pltpu.CompilerParams(use_tc_tiling_on_sc=False),
      scratch_types=[
          pltpu.VMEM((N_CHUNKS, CHUNK), jnp.int32),
          pltpu.VMEM((REL_PER_W,), jnp.int32),
          pltpu.VMEM((CHUNK, DIM), f32),
          pltpu.VMEM((CHUNK, DIM), f32),
          pltpu.VMEM((REL_PER_W, DIM), f32),
          pltpu.VMEM((REL_PER_W, DIM), f32),
          pltpu.SemaphoreType.DMA,
          pltpu.SemaphoreType.DMA,
      ],
  )(_body)
  return run(ent_idx, rel_idx, oe, orl, e, rl)


def kernel(positive, negative, origin_entity_embedding,
           origin_relation_embedding, entity_embedding, relation_embedding):
  tail_idx = jnp.concatenate([positive[:, 2:3], negative], axis=1).reshape(-1)
  ent_idx = jnp.concatenate([positive[:, 0], tail_idx]).reshape(NW, N_CHUNKS, CHUNK)
  rel_idx = positive[:, 1].reshape(NW, REL_PER_W)

  out_oe, out_orl, out_e, out_rl = _gather_all(
      ent_idx.astype(jnp.int32), rel_idx.astype(jnp.int32),
      origin_entity_embedding, origin_relation_embedding,
      entity_embedding, relation_embedding)

  origin_head = out_oe[:BATCH].reshape(BATCH, 1, DIM)
  origin_tail = out_oe[BATCH:].reshape(BATCH, NEG + 1, DIM)
  head = out_e[:BATCH].reshape(BATCH, 1, DIM)
  tail = out_e[BATCH:].reshape(BATCH, NEG + 1, DIM)
  origin_relation = out_orl.reshape(BATCH, 1, DIM)
  relation = out_rl.reshape(BATCH, 1, DIM)
  return (origin_head, origin_relation, origin_tail, head, relation, tail)


# speedup vs baseline: 2.4470x; 1.1010x over previous
import functools

import jax
import jax.numpy as jnp
from jax import lax
from jax.experimental import pallas as pl
from jax.experimental.pallas import tpu as pltpu
from jax.experimental.pallas import tpu_sc as plsc

NC = 2
NS = 16
NW = NC * NS

BATCH = 1024
NEG = 200
DIM = 64

ENT_ROWS = BATCH * (NEG + 2)
REL_ROWS = BATCH
ENT_PER_W = ENT_ROWS // NW
REL_PER_W = REL_ROWS // NW
CHUNK = 101
N_CHUNKS = ENT_PER_W // CHUNK
NB = 6
LEAD = 3


def _body(ent_idx_hbm, rel_idx_hbm, oe_hbm, orl_hbm, e_hbm, rl_hbm,
          out_oe, out_orl, out_e, out_rl,
          idx_v, ridx_v, buf_a, buf_b, rbuf_a, rbuf_b,
          sem_ga, sem_gb, sem_oa, sem_ob, sem_ra, sem_rb):
  wid = lax.axis_index("s") * NC + lax.axis_index("c")

  pltpu.sync_copy(ent_idx_hbm.at[wid], idx_v)
  pltpu.sync_copy(rel_idx_hbm.at[wid], ridx_v)

  cp_ra = pltpu.async_copy(orl_hbm.at[ridx_v], rbuf_a, sem_ra)
  cp_rb = pltpu.async_copy(rl_hbm.at[ridx_v], rbuf_b, sem_rb)

  base = wid * ENT_PER_W

  def fire_gather(t):
    b = lax.rem(t, NB)
    pltpu.async_copy(oe_hbm.at[idx_v.at[t]], buf_a.at[b], sem_ga)
    pltpu.async_copy(e_hbm.at[idx_v.at[t]], buf_b.at[b], sem_gb)

  for t in range(LEAD):
    fire_gather(t)

  def step(t, carry):
    b = lax.rem(t, NB)
    pltpu.make_async_copy(oe_hbm.at[idx_v.at[t]], buf_a.at[b], sem_ga).wait()
    pltpu.make_async_copy(e_hbm.at[idx_v.at[t]], buf_b.at[b], sem_gb).wait()
    off = base + t * CHUNK
    pltpu.async_copy(buf_a.at[b], out_oe.at[pl.ds(off, CHUNK)], sem_oa)
    pltpu.async_copy(buf_b.at[b], out_e.at[pl.ds(off, CHUNK)], sem_ob)

    @pl.when(t >= NB - LEAD)
    def _drain_one():
      pltpu.make_async_copy(buf_a.at[b], out_oe.at[pl.ds(base, CHUNK)], sem_oa).wait()
      pltpu.make_async_copy(buf_b.at[b], out_e.at[pl.ds(base, CHUNK)], sem_ob).wait()

    @pl.when(t + LEAD < N_CHUNKS)
    def _fire_next():
      fire_gather(t + LEAD)

    return carry

  lax.fori_loop(0, N_CHUNKS, step, 0)

  for _ in range(NB - LEAD):
    pltpu.make_async_copy(buf_a.at[0], out_oe.at[pl.ds(base, CHUNK)], sem_oa).wait()
    pltpu.make_async_copy(buf_b.at[0], out_e.at[pl.ds(base, CHUNK)], sem_ob).wait()

  cp_ra.wait()
  cp_rb.wait()
  rbase = wid * REL_PER_W
  pltpu.sync_copy(rbuf_a, out_orl.at[pl.ds(rbase, REL_PER_W)])
  pltpu.sync_copy(rbuf_b, out_rl.at[pl.ds(rbase, REL_PER_W)])


@jax.jit
def _gather_all(ent_idx, rel_idx, oe, orl, e, rl):
  mesh = plsc.VectorSubcoreMesh(core_axis_name="c", subcore_axis_name="s")
  f32 = jnp.float32
  run = functools.partial(
      pl.kernel,
      out_type=(
          jax.ShapeDtypeStruct((ENT_ROWS, DIM), f32),
          jax.ShapeDtypeStruct((REL_ROWS, DIM), f32),
          jax.ShapeDtypeStruct((ENT_ROWS, DIM), f32),
          jax.ShapeDtypeStruct((REL_ROWS, DIM), f32),
      ),
      mesh=mesh,
      compiler_params=pltpu.CompilerParams(use_tc_tiling_on_sc=False),
      scratch_types=[
          pltpu.VMEM((N_CHUNKS, CHUNK), jnp.int32),
          pltpu.VMEM((REL_PER_W,), jnp.int32),
          pltpu.VMEM((NB, CHUNK, DIM), f32),
          pltpu.VMEM((NB, CHUNK, DIM), f32),
          pltpu.VMEM((REL_PER_W, DIM), f32),
          pltpu.VMEM((REL_PER_W, DIM), f32),
          pltpu.SemaphoreType.DMA,
          pltpu.SemaphoreType.DMA,
          pltpu.SemaphoreType.DMA,
          pltpu.SemaphoreType.DMA,
          pltpu.SemaphoreType.DMA,
          pltpu.SemaphoreType.DMA,
      ],
  )(_body)
  return run(ent_idx, rel_idx, oe, orl, e, rl)


def kernel(positive, negative, origin_entity_embedding,
           origin_relation_embedding, entity_embedding, relation_embedding):
  tail_idx = jnp.concatenate([positive[:, 2:3], negative], axis=1).reshape(-1)
  ent_idx = jnp.concatenate([positive[:, 0], tail_idx]).reshape(NW, N_CHUNKS, CHUNK)
  rel_idx = positive[:, 1].reshape(NW, REL_PER_W)

  out_oe, out_orl, out_e, out_rl = _gather_all(
      ent_idx.astype(jnp.int32), rel_idx.astype(jnp.int32),
      origin_entity_embedding, origin_relation_embedding,
      entity_embedding, relation_embedding)

  origin_head = out_oe[:BATCH].reshape(BATCH, 1, DIM)
  origin_tail = out_oe[BATCH:].reshape(BATCH, NEG + 1, DIM)
  head = out_e[:BATCH].reshape(BATCH, 1, DIM)
  tail = out_e[BATCH:].reshape(BATCH, NEG + 1, DIM)
  origin_relation = out_orl.reshape(BATCH, 1, DIM)
  relation = out_rl.reshape(BATCH, 1, DIM)
  return (origin_head, origin_relation, origin_tail, head, relation, tail)


# speedup vs baseline: 4.0950x; 1.6735x over previous
import functools

import jax
import jax.numpy as jnp
from jax import lax
from jax.experimental import pallas as pl
from jax.experimental.pallas import tpu as pltpu
from jax.experimental.pallas import tpu_sc as plsc

NC = 2
NS = 16
NW = NC * NS

BATCH = 1024
NEG = 200
DIM = 64

B_PER_W = BATCH // NW
NEG_PER_W = B_PER_W * NEG
CHUNK = NEG // 2
N_CHUNKS = NEG_PER_W // CHUNK
NB = 6
LEAD = 3


def _body(aux_hbm, neg_hbm, oe_hbm, orl_hbm, e_hbm, rl_hbm,
          out_oh, out_orl, out_ot, out_h, out_rl, out_t,
          aux_v, nidx_v, buf_a, buf_b, sbuf, sem_ga, sem_gb, sem_oa, sem_ob,
          sem_s):
  wid = lax.axis_index("s") * NC + lax.axis_index("c")
  b0 = wid * B_PER_W

  pltpu.sync_copy(aux_hbm.at[wid], aux_v)
  pltpu.sync_copy(neg_hbm.at[wid], nidx_v)

  small = (
      (oe_hbm, aux_v.at[0]), (e_hbm, aux_v.at[0]),
      (orl_hbm, aux_v.at[1]), (rl_hbm, aux_v.at[1]),
      (oe_hbm, aux_v.at[2]), (e_hbm, aux_v.at[2]),
  )
  for i, (tbl, idx) in enumerate(small):
    pltpu.async_copy(tbl.at[idx], sbuf.at[i], sem_s)

  def fire_gather(t):
    b = lax.rem(t, NB)
    pltpu.async_copy(oe_hbm.at[nidx_v.at[t]], buf_a.at[b], sem_ga)
    pltpu.async_copy(e_hbm.at[nidx_v.at[t]], buf_b.at[b], sem_gb)

  for t in range(LEAD):
    fire_gather(t)

  def step(t, carry):
    b = lax.rem(t, NB)
    pltpu.make_async_copy(oe_hbm.at[nidx_v.at[t]], buf_a.at[b], sem_ga).wait()
    pltpu.make_async_copy(e_hbm.at[nidx_v.at[t]], buf_b.at[b], sem_gb).wait()
    row = b0 + lax.div(t, 2)
    col = 1 + lax.rem(t, 2) * CHUNK
    pltpu.async_copy(buf_a.at[b], out_ot.at[row, pl.ds(col, CHUNK)], sem_oa)
    pltpu.async_copy(buf_b.at[b], out_t.at[row, pl.ds(col, CHUNK)], sem_ob)

    @pl.when(t >= NB - LEAD)
    def _drain_one():
      pltpu.make_async_copy(buf_a.at[b], out_ot.at[0, pl.ds(1, CHUNK)], sem_oa).wait()
      pltpu.make_async_copy(buf_b.at[b], out_t.at[0, pl.ds(1, CHUNK)], sem_ob).wait()

    @pl.when(t + LEAD < N_CHUNKS)
    def _fire_next():
      fire_gather(t + LEAD)

    return carry

  lax.fori_loop(0, N_CHUNKS, step, 0)

  for _ in range(NB - LEAD):
    pltpu.make_async_copy(buf_a.at[0], out_ot.at[0, pl.ds(1, CHUNK)], sem_oa).wait()
    pltpu.make_async_copy(buf_b.at[0], out_t.at[0, pl.ds(1, CHUNK)], sem_ob).wait()

  for i, (tbl, idx) in enumerate(small):
    pltpu.make_async_copy(tbl.at[idx], sbuf.at[i], sem_s).wait()
  pltpu.sync_copy(sbuf.at[0], out_oh.at[pl.ds(b0, B_PER_W)])
  pltpu.sync_copy(sbuf.at[1], out_h.at[pl.ds(b0, B_PER_W)])
  pltpu.sync_copy(sbuf.at[2], out_orl.at[pl.ds(b0, B_PER_W)])
  pltpu.sync_copy(sbuf.at[3], out_rl.at[pl.ds(b0, B_PER_W)])
  pltpu.sync_copy(sbuf.at[4], out_ot.at[pl.ds(b0, B_PER_W), 0])
  pltpu.sync_copy(sbuf.at[5], out_t.at[pl.ds(b0, B_PER_W), 0])


@jax.jit
def _gather_all(aux_idx, neg_idx, oe, orl, e, rl):
  mesh = plsc.VectorSubcoreMesh(core_axis_name="c", subcore_axis_name="s")
  f32 = jnp.float32
  run = functools.partial(
      pl.kernel,
      out_type=(
          jax.ShapeDtypeStruct((BATCH, DIM), f32),
          jax.ShapeDtypeStruct((BATCH, DIM), f32),
          jax.ShapeDtypeStruct((BATCH, NEG + 1, DIM), f32),
          jax.ShapeDtypeStruct((BATCH, DIM), f32),
          jax.ShapeDtypeStruct((BATCH, DIM), f32),
          jax.ShapeDtypeStruct((BATCH, NEG + 1, DIM), f32),
      ),
      mesh=mesh,
      compiler_params=pltpu.CompilerParams(use_tc_tiling_on_sc=False),
      scratch_types=[
          pltpu.VMEM((3, B_PER_W), jnp.int32),
          pltpu.VMEM((N_CHUNKS, CHUNK), jnp.int32),
          pltpu.VMEM((NB, CHUNK, DIM), f32),
          pltpu.VMEM((NB, CHUNK, DIM), f32),
          pltpu.VMEM((6, B_PER_W, DIM), f32),
          pltpu.SemaphoreType.DMA,
          pltpu.SemaphoreType.DMA,
          pltpu.SemaphoreType.DMA,
          pltpu.SemaphoreType.DMA,
          pltpu.SemaphoreType.DMA,
      ],
  )(_body)
  return run(aux_idx, neg_idx, oe, orl, e, rl)


def kernel(positive, negative, origin_entity_embedding,
           origin_relation_embedding, entity_embedding, relation_embedding):
  aux_idx = positive.reshape(NW, B_PER_W, 3).transpose(0, 2, 1).astype(jnp.int32)
  neg_idx = negative.reshape(NW, N_CHUNKS, CHUNK).astype(jnp.int32)

  out_oh, out_orl, out_ot, out_h, out_rl, out_t = _gather_all(
      aux_idx, neg_idx,
      origin_entity_embedding, origin_relation_embedding,
      entity_embedding, relation_embedding)

  return (out_oh.reshape(BATCH, 1, DIM), out_orl.reshape(BATCH, 1, DIM),
          out_ot, out_h.reshape(BATCH, 1, DIM), out_rl.reshape(BATCH, 1, DIM),
          out_t)
